# baseline (speedup 1.0000x reference)
import functools

import jax
import jax.numpy as jnp
from jax import lax
from jax.experimental import pallas as pl
from jax.experimental.pallas import tpu as pltpu
from jax.experimental.pallas import tpu_sc as plsc

NUM_STATES = 100000
NUM_ACTIONS = 10
EMBED_DIM = 64
FOURIER_DIM = 64
BATCH = 16384

NUM_CORES = 2
NUM_SUBCORES = 16
NUM_WORKERS = NUM_CORES * NUM_SUBCORES
ROWS_PER_WORKER = BATCH // NUM_WORKERS
CHUNK = 128
NCHUNK = ROWS_PER_WORKER // CHUNK
LANES = 16

BB = 2048
GRID = BATCH // BB


def _sc_gather_body(obs_hbm, nobs_hbm, act_hbm, semb_hbm, saemb_hbm,
                    embed_out, sa_out,
                    idx1_v, idx2_v, act_v, rows1_v, rows2_v, sem1, sem2):
    wid = lax.axis_index("s") * NUM_CORES + lax.axis_index("c")
    crow = wid * NCHUNK
    base = wid * ROWS_PER_WORKER

    pltpu.sync_copy(obs_hbm.at[pl.ds(crow, NCHUNK)], idx1_v)
    pltpu.sync_copy(nobs_hbm.at[pl.ds(crow, NCHUNK)], idx2_v)
    pltpu.sync_copy(act_hbm.at[pl.ds(crow, NCHUNK)], act_v)

    for j in range(NCHUNK):
        @pl.loop(0, CHUNK, step=LANES)
        def _(k, j=j):
            idx2_v[j, pl.ds(k, LANES)] = (
                idx2_v[j, pl.ds(k, LANES)] * NUM_ACTIONS
                + act_v[j, pl.ds(k, LANES)])

    copies = []
    for j in range(NCHUNK):
        copies.append(pltpu.async_copy(
            semb_hbm.at[idx1_v.at[j]],
            rows1_v.at[pl.ds(j * CHUNK, CHUNK)], sem1))
        copies.append(pltpu.async_copy(
            saemb_hbm.at[idx2_v.at[j]],
            rows2_v.at[pl.ds(j * CHUNK, CHUNK)], sem2))
    for c in copies:
        c.wait()

    pltpu.sync_copy(rows1_v, embed_out.at[pl.ds(base, ROWS_PER_WORKER)])
    pltpu.sync_copy(rows2_v, sa_out.at[pl.ds(base, ROWS_PER_WORKER)])


def _dense_body(emb_ref, sa_ref, act_ref, omega_ref, shift_ref,
                aemb_ref, asq_ref, pol_ref, out_ref):
    emb = emb_ref[...]
    sa = sa_ref[...]
    omega = omega_ref[...]
    shift = shift_ref[...]
    aemb = aemb_ref[...]
    asq = asq_ref[...]
    pol = pol_ref[...]
    act = act_ref[0, 0, :]

    def dot_bf16(a, b, dims):
        return lax.dot_general(a.astype(jnp.bfloat16),
                               b.astype(jnp.bfloat16), dims,
                               preferred_element_type=jnp.float32)

    stddev = jnp.sqrt(jnp.maximum(1e-8, asq - aemb * aemb))
    nomega = omega / stddev
    proj = dot_bf16(emb - aemb, nomega, (((1,), (1,)), ((), ())))
    proj = proj * (1.0 / (EMBED_DIM ** 0.5))
    el = jnp.cos(proj + shift)
    logits = dot_bf16(el, pol, (((1,), (0,)), ((), ())))
    m = jnp.max(logits, axis=-1, keepdims=True)
    e = jnp.exp(logits - m)
    probs = e / jnp.sum(e, axis=-1, keepdims=True)
    onehot = lax.broadcasted_iota(jnp.int32, (BB, NUM_ACTIONS), 1) \
        == act[:, None]
    ap = jnp.sum(jnp.where(onehot, probs, 0.0), axis=-1)
    ne = jnp.cos(dot_bf16(sa, omega, (((1,), (1,)), ((), ()))) + shift)
    et = ((2.0 / FOURIER_DIM) ** 0.5) * jnp.sum(el * ne, axis=-1)
    out_ref[...] = jnp.concatenate(
        [probs, ap[:, None], et[:, None]], axis=-1)


def _sc_call(observation, action, next_observation, state_embedder,
             state_action_embedder):
    obs2 = observation.reshape(BATCH // CHUNK, CHUNK)
    nobs2 = next_observation.reshape(BATCH // CHUNK, CHUNK)
    act2 = action.reshape(BATCH // CHUNK, CHUNK)
    sa_flat = state_action_embedder.reshape(NUM_STATES * NUM_ACTIONS,
                                            EMBED_DIM)

    mesh = plsc.VectorSubcoreMesh(core_axis_name="c", subcore_axis_name="s")
    sc_gather = pl.kernel(
        _sc_gather_body,
        out_type=[jax.ShapeDtypeStruct((BATCH, EMBED_DIM), jnp.float32),
                  jax.ShapeDtypeStruct((BATCH, EMBED_DIM), jnp.float32)],
        mesh=mesh,
        scratch_types=[
            pltpu.VMEM((NCHUNK, CHUNK), jnp.int32),
            pltpu.VMEM((NCHUNK, CHUNK), jnp.int32),
            pltpu.VMEM((NCHUNK, CHUNK), jnp.int32),
            pltpu.VMEM((ROWS_PER_WORKER, EMBED_DIM), jnp.float32),
            pltpu.VMEM((ROWS_PER_WORKER, EMBED_DIM), jnp.float32),
            pltpu.SemaphoreType.DMA,
            pltpu.SemaphoreType.DMA,
        ],
        compiler_params=pltpu.CompilerParams(use_tc_tiling_on_sc=False),
    )
    return sc_gather(obs2, nobs2, act2, state_embedder, sa_flat)


def kernel(observation, action, next_observation, state_embedder,
           state_action_embedder, omega, shift, average_embed,
           average_square, embed_policy):
    embed, sa_embed = _sc_call(observation, action, next_observation,
                               state_embedder, state_action_embedder)

    act3 = action.reshape(GRID, 1, BB)
    out = pl.pallas_call(
        _dense_body,
        grid=(GRID,),
        in_specs=[
            pl.BlockSpec((BB, EMBED_DIM), lambda i: (i, 0)),
            pl.BlockSpec((BB, EMBED_DIM), lambda i: (i, 0)),
            pl.BlockSpec((1, 1, BB), lambda i: (i, 0, 0)),
            pl.BlockSpec((FOURIER_DIM, EMBED_DIM), lambda i: (0, 0)),
            pl.BlockSpec((1, FOURIER_DIM), lambda i: (0, 0)),
            pl.BlockSpec((1, EMBED_DIM), lambda i: (0, 0)),
            pl.BlockSpec((1, EMBED_DIM), lambda i: (0, 0)),
            pl.BlockSpec((FOURIER_DIM, NUM_ACTIONS), lambda i: (0, 0)),
        ],
        out_specs=pl.BlockSpec((BB, NUM_ACTIONS + 2), lambda i: (i, 0)),
        out_shape=jax.ShapeDtypeStruct((BATCH, NUM_ACTIONS + 2),
                                       jnp.float32),
    )(embed, sa_embed, act3, omega, shift.reshape(1, FOURIER_DIM),
      average_embed.reshape(1, EMBED_DIM), average_square.reshape(1, EMBED_DIM),
      embed_policy)
    return out


# speedup vs baseline: 1.4043x; 1.4043x over previous
import functools

import jax
import jax.numpy as jnp
from jax import lax
from jax.experimental import pallas as pl
from jax.experimental.pallas import tpu as pltpu
from jax.experimental.pallas import tpu_sc as plsc

NUM_STATES = 100000
NUM_ACTIONS = 10
EMBED_DIM = 64
FOURIER_DIM = 64
BATCH = 16384

NUM_CORES = 2
NUM_SUBCORES = 16
NUM_WORKERS = NUM_CORES * NUM_SUBCORES
ROWS_PER_WORKER = BATCH // NUM_WORKERS
CHUNK = 128
NCHUNK = ROWS_PER_WORKER // CHUNK
LANES = 16

S_BLK = 1024
N_SBLK = 49
PAIR = N_SBLK * S_BLK

BB = 2048
GRID = BATCH // BB


def _fmt1_body(lo_ref, hi_ref, out_ref):
    lo = jnp.transpose(lo_ref[...], (1, 0))
    hi = jnp.transpose(hi_ref[...], (1, 0))
    out_ref[...] = jnp.concatenate([lo, hi], axis=1)


def _fmt2_body(lo_ref, hi_ref, out_ref):
    lo = jnp.transpose(lo_ref[...], (1, 0))
    hi = jnp.transpose(hi_ref[...], (1, 0))
    out_ref[...] = jnp.concatenate([lo, hi], axis=1)[None]


def _sc_gather1_body(obs_hbm, fmt1_hbm, out_hbm, idx_v, rows_v, sem):
    wid = lax.axis_index("s") * NUM_CORES + lax.axis_index("c")
    crow = wid * NCHUNK
    base = wid * ROWS_PER_WORKER
    pltpu.sync_copy(obs_hbm.at[pl.ds(crow, NCHUNK)], idx_v)
    for j in range(NCHUNK):
        @pl.loop(0, CHUNK, step=LANES)
        def _(k, j=j):
            v = idx_v[j, pl.ds(k, LANES)]
            idx_v[j, pl.ds(k, LANES)] = jnp.where(v >= PAIR, v - PAIR, v)
    copies = []
    for j in range(NCHUNK):
        copies.append(pltpu.async_copy(
            fmt1_hbm.at[idx_v.at[j]],
            rows_v.at[pl.ds(j * CHUNK, CHUNK)], sem))
    for c in copies:
        c.wait()
    pltpu.sync_copy(rows_v, out_hbm.at[pl.ds(base, ROWS_PER_WORKER)])


def _sc_gather2_body(nobs_hbm, act_hbm, fmt2_hbm, out_hbm,
                     idx_v, act_v, rows_v, sem):
    wid = lax.axis_index("s") * NUM_CORES + lax.axis_index("c")
    crow = wid * NCHUNK
    base = wid * ROWS_PER_WORKER
    pltpu.sync_copy(nobs_hbm.at[pl.ds(crow, NCHUNK)], idx_v)
    pltpu.sync_copy(act_hbm.at[pl.ds(crow, NCHUNK)], act_v)
    for j in range(NCHUNK):
        @pl.loop(0, CHUNK, step=LANES)
        def _(k, j=j):
            v = idx_v[j, pl.ds(k, LANES)]
            v = jnp.where(v >= PAIR, v - PAIR, v)
            idx_v[j, pl.ds(k, LANES)] = (
                act_v[j, pl.ds(k, LANES)] * PAIR + v)
    copies = []
    for j in range(NCHUNK):
        copies.append(pltpu.async_copy(
            fmt2_hbm.at[idx_v.at[j]],
            rows_v.at[pl.ds(j * CHUNK, CHUNK)], sem))
    for c in copies:
        c.wait()
    pltpu.sync_copy(rows_v, out_hbm.at[pl.ds(base, ROWS_PER_WORKER)])


def _dense_body(emb2_ref, sa2_ref, obs_ref, act_ref, nobs_ref, omega_ref,
                shift_ref, aemb_ref, asq_ref, pol_ref, out_ref):
    emb2 = emb2_ref[...]
    sa2 = sa2_ref[...]
    omega = omega_ref[...]
    shift = shift_ref[...]
    aemb = aemb_ref[...]
    asq = asq_ref[...]
    pol = pol_ref[...]
    obs = obs_ref[0, 0, :]
    act = act_ref[0, 0, :]
    nobs = nobs_ref[0, 0, :]

    half1 = obs[:, None] >= PAIR
    emb = jnp.where(half1, emb2[:, EMBED_DIM:], emb2[:, :EMBED_DIM])
    half2 = nobs[:, None] >= PAIR
    sa = jnp.where(half2, sa2[:, EMBED_DIM:], sa2[:, :EMBED_DIM])

    def dot_bf16(a, b, dims):
        return lax.dot_general(a.astype(jnp.bfloat16),
                               b.astype(jnp.bfloat16), dims,
                               preferred_element_type=jnp.float32)

    stddev = jnp.sqrt(jnp.maximum(1e-8, asq - aemb * aemb))
    nomega = omega / stddev
    proj = dot_bf16(emb - aemb, nomega, (((1,), (1,)), ((), ())))
    proj = proj * (1.0 / (EMBED_DIM ** 0.5))
    el = jnp.cos(proj + shift)
    logits = dot_bf16(el, pol, (((1,), (0,)), ((), ())))
    m = jnp.max(logits, axis=-1, keepdims=True)
    e = jnp.exp(logits - m)
    probs = e / jnp.sum(e, axis=-1, keepdims=True)
    onehot = lax.broadcasted_iota(jnp.int32, (BB, NUM_ACTIONS), 1) \
        == act[:, None]
    ap = jnp.sum(jnp.where(onehot, probs, 0.0), axis=-1)
    ne = jnp.cos(dot_bf16(sa, omega, (((1,), (1,)), ((), ()))) + shift)
    et = ((2.0 / FOURIER_DIM) ** 0.5) * jnp.sum(el * ne, axis=-1)
    out_ref[...] = jnp.concatenate(
        [probs, ap[:, None], et[:, None]], axis=-1)


def kernel(observation, action, next_observation, state_embedder,
           state_action_embedder, omega, shift, average_embed,
           average_square, embed_policy):
    semb_t = jnp.transpose(state_embedder, (1, 0))
    sat_t = jnp.transpose(state_action_embedder, (1, 2, 0)) \
        .reshape(NUM_ACTIONS * EMBED_DIM, NUM_STATES)

    fmt1 = pl.pallas_call(
        _fmt1_body,
        grid=(N_SBLK,),
        in_specs=[pl.BlockSpec((EMBED_DIM, S_BLK), lambda c: (0, c)),
                  pl.BlockSpec((EMBED_DIM, S_BLK), lambda c: (0, N_SBLK + c))],
        out_specs=pl.BlockSpec((S_BLK, 2 * EMBED_DIM), lambda c: (c, 0)),
        out_shape=jax.ShapeDtypeStruct((PAIR, 2 * EMBED_DIM), jnp.float32),
    )(semb_t, semb_t)

    fmt2 = pl.pallas_call(
        _fmt2_body,
        grid=(NUM_ACTIONS, N_SBLK),
        in_specs=[pl.BlockSpec((EMBED_DIM, S_BLK), lambda a, c: (a, c)),
                  pl.BlockSpec((EMBED_DIM, S_BLK),
                               lambda a, c: (a, N_SBLK + c))],
        out_specs=pl.BlockSpec((1, S_BLK, 2 * EMBED_DIM),
                               lambda a, c: (a, c, 0)),
        out_shape=jax.ShapeDtypeStruct(
            (NUM_ACTIONS, PAIR, 2 * EMBED_DIM), jnp.float32),
    )(sat_t, sat_t)
    fmt2 = fmt2.reshape(NUM_ACTIONS * PAIR, 2 * EMBED_DIM)

    obs2 = observation.reshape(BATCH // CHUNK, CHUNK)
    nobs2 = next_observation.reshape(BATCH // CHUNK, CHUNK)
    act2 = action.reshape(BATCH // CHUNK, CHUNK)

    mesh = plsc.VectorSubcoreMesh(core_axis_name="c", subcore_axis_name="s")
    embed2 = pl.kernel(
        _sc_gather1_body,
        out_type=jax.ShapeDtypeStruct((BATCH, 2 * EMBED_DIM), jnp.float32),
        mesh=mesh,
        scratch_types=[
            pltpu.VMEM((NCHUNK, CHUNK), jnp.int32),
            pltpu.VMEM((ROWS_PER_WORKER, 2 * EMBED_DIM), jnp.float32),
            pltpu.SemaphoreType.DMA,
        ],
    )(obs2, fmt1)

    sa2 = pl.kernel(
        _sc_gather2_body,
        out_type=jax.ShapeDtypeStruct((BATCH, 2 * EMBED_DIM), jnp.float32),
        mesh=mesh,
        scratch_types=[
            pltpu.VMEM((NCHUNK, CHUNK), jnp.int32),
            pltpu.VMEM((NCHUNK, CHUNK), jnp.int32),
            pltpu.VMEM((ROWS_PER_WORKER, 2 * EMBED_DIM), jnp.float32),
            pltpu.SemaphoreType.DMA,
        ],
    )(nobs2, act2, fmt2)

    obs3 = observation.reshape(GRID, 1, BB)
    act3 = action.reshape(GRID, 1, BB)
    nobs3 = next_observation.reshape(GRID, 1, BB)
    out = pl.pallas_call(
        _dense_body,
        grid=(GRID,),
        in_specs=[
            pl.BlockSpec((BB, 2 * EMBED_DIM), lambda i: (i, 0)),
            pl.BlockSpec((BB, 2 * EMBED_DIM), lambda i: (i, 0)),
            pl.BlockSpec((1, 1, BB), lambda i: (i, 0, 0)),
            pl.BlockSpec((1, 1, BB), lambda i: (i, 0, 0)),
            pl.BlockSpec((1, 1, BB), lambda i: (i, 0, 0)),
            pl.BlockSpec((FOURIER_DIM, EMBED_DIM), lambda i: (0, 0)),
            pl.BlockSpec((1, FOURIER_DIM), lambda i: (0, 0)),
            pl.BlockSpec((1, EMBED_DIM), lambda i: (0, 0)),
            pl.BlockSpec((1, EMBED_DIM), lambda i: (0, 0)),
            pl.BlockSpec((FOURIER_DIM, NUM_ACTIONS), lambda i: (0, 0)),
        ],
        out_specs=pl.BlockSpec((BB, NUM_ACTIONS + 2), lambda i: (i, 0)),
        out_shape=jax.ShapeDtypeStruct((BATCH, NUM_ACTIONS + 2),
                                       jnp.float32),
    )(embed2, sa2, obs3, act3, nobs3, omega, shift.reshape(1, FOURIER_DIM),
      average_embed.reshape(1, EMBED_DIM), average_square.reshape(1, EMBED_DIM),
      embed_policy)
    return out


# speedup vs baseline: 1.4892x; 1.0605x over previous
import functools

import jax
import jax.numpy as jnp
from jax import lax
from jax.experimental import pallas as pl
from jax.experimental.pallas import tpu as pltpu
from jax.experimental.pallas import tpu_sc as plsc

NUM_STATES = 100000
NUM_ACTIONS = 10
EMBED_DIM = 64
FOURIER_DIM = 64
BATCH = 16384

NUM_CORES = 2
NUM_SUBCORES = 16
NUM_WORKERS = NUM_CORES * NUM_SUBCORES
ROWS_PER_WORKER = BATCH // NUM_WORKERS
CHUNK = 128
NCHUNK = ROWS_PER_WORKER // CHUNK
LANES = 16

S_BLK = 1024
N_SBLK = 49
PAIR = N_SBLK * S_BLK

BB = 2048
GRID = BATCH // BB


def _dot_t(x, w):
    return lax.dot_general(x.astype(jnp.bfloat16), w.astype(jnp.bfloat16),
                           (((0,), (1,)), ((), ())),
                           preferred_element_type=jnp.float32)


def _fmt1_body(lo_ref, hi_ref, omega_ref, shift_ref, aemb_ref, asq_ref,
               out_ref):
    aemb = aemb_ref[...]
    stddev = jnp.sqrt(jnp.maximum(1e-8, asq_ref[...] - aemb * aemb))
    lo = (lo_ref[...] - aemb) / stddev
    hi = (hi_ref[...] - aemb) / stddev
    shift = shift_ref[...]
    plo = _dot_t(lo, omega_ref[...]) * (1.0 / (EMBED_DIM ** 0.5)) + shift
    phi = _dot_t(hi, omega_ref[...]) * (1.0 / (EMBED_DIM ** 0.5)) + shift
    out_ref[...] = jnp.concatenate([plo, phi], axis=1)


def _fmt2_body(lo_ref, hi_ref, omega_ref, shift_ref, out_ref):
    shift = shift_ref[...]
    plo = _dot_t(lo_ref[...], omega_ref[...]) + shift
    phi = _dot_t(hi_ref[...], omega_ref[...]) + shift
    out_ref[...] = jnp.concatenate([plo, phi], axis=1)[None]


def _sc_gather1_body(obs_hbm, fmt1_hbm, out_hbm, idx_v, rows_v, sem):
    wid = lax.axis_index("s") * NUM_CORES + lax.axis_index("c")
    crow = wid * NCHUNK
    base = wid * ROWS_PER_WORKER
    pltpu.sync_copy(obs_hbm.at[pl.ds(crow, NCHUNK)], idx_v)
    for j in range(NCHUNK):
        @pl.loop(0, CHUNK, step=LANES)
        def _(k, j=j):
            v = idx_v[j, pl.ds(k, LANES)]
            idx_v[j, pl.ds(k, LANES)] = jnp.where(v >= PAIR, v - PAIR, v)
    copies = []
    for j in range(NCHUNK):
        copies.append(pltpu.async_copy(
            fmt1_hbm.at[idx_v.at[j]],
            rows_v.at[pl.ds(j * CHUNK, CHUNK)], sem))
    for c in copies:
        c.wait()
    pltpu.sync_copy(rows_v, out_hbm.at[pl.ds(base, ROWS_PER_WORKER)])


def _sc_gather2_body(nobs_hbm, act_hbm, fmt2_hbm, out_hbm,
                     idx_v, act_v, rows_v, sem):
    wid = lax.axis_index("s") * NUM_CORES + lax.axis_index("c")
    crow = wid * NCHUNK
    base = wid * ROWS_PER_WORKER
    pltpu.sync_copy(nobs_hbm.at[pl.ds(crow, NCHUNK)], idx_v)
    pltpu.sync_copy(act_hbm.at[pl.ds(crow, NCHUNK)], act_v)
    for j in range(NCHUNK):
        @pl.loop(0, CHUNK, step=LANES)
        def _(k, j=j):
            v = idx_v[j, pl.ds(k, LANES)]
            v = jnp.where(v >= PAIR, v - PAIR, v)
            idx_v[j, pl.ds(k, LANES)] = (
                act_v[j, pl.ds(k, LANES)] * PAIR + v)
    copies = []
    for j in range(NCHUNK):
        copies.append(pltpu.async_copy(
            fmt2_hbm.at[idx_v.at[j]],
            rows_v.at[pl.ds(j * CHUNK, CHUNK)], sem))
    for c in copies:
        c.wait()
    pltpu.sync_copy(rows_v, out_hbm.at[pl.ds(base, ROWS_PER_WORKER)])


def _dense_body(pe2_ref, pn2_ref, obs_ref, act_ref, nobs_ref, pol_ref,
                out_ref):
    pe2 = pe2_ref[...]
    pn2 = pn2_ref[...]
    pol = pol_ref[...]
    obs = obs_ref[0, 0, :]
    act = act_ref[0, 0, :]
    nobs = nobs_ref[0, 0, :]

    half1 = obs[:, None] >= PAIR
    pe = jnp.where(half1, pe2[:, FOURIER_DIM:], pe2[:, :FOURIER_DIM])
    half2 = nobs[:, None] >= PAIR
    pn = jnp.where(half2, pn2[:, FOURIER_DIM:], pn2[:, :FOURIER_DIM])

    def dot_bf16(a, b, dims):
        return lax.dot_general(a.astype(jnp.bfloat16),
                               b.astype(jnp.bfloat16), dims,
                               preferred_element_type=jnp.float32)

    el = jnp.cos(pe)
    logits = dot_bf16(el, pol, (((1,), (0,)), ((), ())))
    m = jnp.max(logits, axis=-1, keepdims=True)
    e = jnp.exp(logits - m)
    probs = e / jnp.sum(e, axis=-1, keepdims=True)
    onehot = lax.broadcasted_iota(jnp.int32, (BB, NUM_ACTIONS), 1) \
        == act[:, None]
    ap = jnp.sum(jnp.where(onehot, probs, 0.0), axis=-1)
    ne = jnp.cos(pn)
    et = ((2.0 / FOURIER_DIM) ** 0.5) * jnp.sum(el * ne, axis=-1)
    out_ref[...] = jnp.concatenate(
        [probs, ap[:, None], et[:, None]], axis=-1)


def kernel(observation, action, next_observation, state_embedder,
           state_action_embedder, omega, shift, average_embed,
           average_square, embed_policy):
    semb_t = jnp.transpose(state_embedder, (1, 0))
    sat_t = jnp.transpose(state_action_embedder, (1, 2, 0)) \
        .reshape(NUM_ACTIONS * EMBED_DIM, NUM_STATES)

    shift_row = shift.reshape(1, FOURIER_DIM)
    aemb_col = average_embed.reshape(EMBED_DIM, 1)
    asq_col = average_square.reshape(EMBED_DIM, 1)
    cfull = lambda c: (0, 0)
    fmt1 = pl.pallas_call(
        _fmt1_body,
        grid=(N_SBLK,),
        in_specs=[pl.BlockSpec((EMBED_DIM, S_BLK), lambda c: (0, c)),
                  pl.BlockSpec((EMBED_DIM, S_BLK), lambda c: (0, N_SBLK + c)),
                  pl.BlockSpec((FOURIER_DIM, EMBED_DIM), cfull),
                  pl.BlockSpec((1, FOURIER_DIM), cfull),
                  pl.BlockSpec((EMBED_DIM, 1), cfull),
                  pl.BlockSpec((EMBED_DIM, 1), cfull)],
        out_specs=pl.BlockSpec((S_BLK, 2 * FOURIER_DIM), lambda c: (c, 0)),
        out_shape=jax.ShapeDtypeStruct((PAIR, 2 * FOURIER_DIM), jnp.float32),
    )(semb_t, semb_t, omega, shift_row, aemb_col, asq_col)

    acfull = lambda a, c: (0, 0)
    fmt2 = pl.pallas_call(
        _fmt2_body,
        grid=(NUM_ACTIONS, N_SBLK),
        in_specs=[pl.BlockSpec((EMBED_DIM, S_BLK), lambda a, c: (a, c)),
                  pl.BlockSpec((EMBED_DIM, S_BLK),
                               lambda a, c: (a, N_SBLK + c)),
                  pl.BlockSpec((FOURIER_DIM, EMBED_DIM), acfull),
                  pl.BlockSpec((1, FOURIER_DIM), acfull)],
        out_specs=pl.BlockSpec((1, S_BLK, 2 * FOURIER_DIM),
                               lambda a, c: (a, c, 0)),
        out_shape=jax.ShapeDtypeStruct(
            (NUM_ACTIONS, PAIR, 2 * FOURIER_DIM), jnp.float32),
    )(sat_t, sat_t, omega, shift_row)
    fmt2 = fmt2.reshape(NUM_ACTIONS * PAIR, 2 * FOURIER_DIM)

    obs2 = observation.reshape(BATCH // CHUNK, CHUNK)
    nobs2 = next_observation.reshape(BATCH // CHUNK, CHUNK)
    act2 = action.reshape(BATCH // CHUNK, CHUNK)

    mesh = plsc.VectorSubcoreMesh(core_axis_name="c", subcore_axis_name="s")
    embed2 = pl.kernel(
        _sc_gather1_body,
        out_type=jax.ShapeDtypeStruct((BATCH, 2 * EMBED_DIM), jnp.float32),
        mesh=mesh,
        scratch_types=[
            pltpu.VMEM((NCHUNK, CHUNK), jnp.int32),
            pltpu.VMEM((ROWS_PER_WORKER, 2 * EMBED_DIM), jnp.float32),
            pltpu.SemaphoreType.DMA,
        ],
    )(obs2, fmt1)

    sa2 = pl.kernel(
        _sc_gather2_body,
        out_type=jax.ShapeDtypeStruct((BATCH, 2 * EMBED_DIM), jnp.float32),
        mesh=mesh,
        scratch_types=[
            pltpu.VMEM((NCHUNK, CHUNK), jnp.int32),
            pltpu.VMEM((NCHUNK, CHUNK), jnp.int32),
            pltpu.VMEM((ROWS_PER_WORKER, 2 * EMBED_DIM), jnp.float32),
            pltpu.SemaphoreType.DMA,
        ],
    )(nobs2, act2, fmt2)

    obs3 = observation.reshape(GRID, 1, BB)
    act3 = action.reshape(GRID, 1, BB)
    nobs3 = next_observation.reshape(GRID, 1, BB)
    out = pl.pallas_call(
        _dense_body,
        grid=(GRID,),
        in_specs=[
            pl.BlockSpec((BB, 2 * FOURIER_DIM), lambda i: (i, 0)),
            pl.BlockSpec((BB, 2 * FOURIER_DIM), lambda i: (i, 0)),
            pl.BlockSpec((1, 1, BB), lambda i: (i, 0, 0)),
            pl.BlockSpec((1, 1, BB), lambda i: (i, 0, 0)),
            pl.BlockSpec((1, 1, BB), lambda i: (i, 0, 0)),
            pl.BlockSpec((FOURIER_DIM, NUM_ACTIONS), lambda i: (0, 0)),
        ],
        out_specs=pl.BlockSpec((BB, NUM_ACTIONS + 2), lambda i: (i, 0)),
        out_shape=jax.ShapeDtypeStruct((BATCH, NUM_ACTIONS + 2),
                                       jnp.float32),
    )(embed2, sa2, obs3, act3, nobs3, embed_policy)
    return out
